# Initial kernel scaffold; baseline (speedup 1.0000x reference)
import jax
import jax.numpy as jnp
from jax.experimental import pallas as pl


def kernel(node_feat, node_graph_ids, edge_feat, edge_graph_ids, num_graphs):
    raise NotImplementedError("write your pallas kernel here")


# speedup vs baseline: 4.6940x; 4.6940x over previous
import functools

import jax
import jax.numpy as jnp
from jax import lax
from jax.experimental import pallas as pl
from jax.experimental.pallas import tpu as pltpu
from jax.experimental.pallas import tpu_sc as plsc

N_NODES, D_N = 10000, 128
N_EDGES, D_E = 320000, 16
G = 64
NC, NS = 2, 16
NW = NC * NS
LANES = 16
NODE_CHUNK = 304
NODE_TAIL = N_NODES - NW * NODE_CHUNK
NODE_TAIL_GROUPS = NODE_TAIL // LANES
EDGE_PER_W = N_EDGES // NW
E_CHUNK = 2000
N_ECHUNKS = EDGE_PER_W // E_CHUNK


def _sc_partials(node_feat, node_ids, edge_feat, edge_ids):
    mesh = plsc.VectorSubcoreMesh(core_axis_name="c", subcore_axis_name="s")

    @functools.partial(
        pl.kernel,
        out_type=(
            jax.ShapeDtypeStruct((NW, G, D_N), jnp.float32),
            jax.ShapeDtypeStruct((NW, G, D_E), jnp.float32),
        ),
        mesh=mesh,
        compiler_params=pltpu.CompilerParams(use_tc_tiling_on_sc=False),
        scratch_types=[
            pltpu.VMEM((NODE_CHUNK, D_N), jnp.float32),
            pltpu.VMEM((NODE_CHUNK,), jnp.int32),
            pltpu.VMEM((LANES, D_N), jnp.float32),
            pltpu.VMEM((LANES,), jnp.int32),
            pltpu.VMEM((E_CHUNK, D_E), jnp.float32),
            pltpu.VMEM((EDGE_PER_W,), jnp.int32),
            pltpu.VMEM((G, D_N), jnp.float32),
            pltpu.VMEM((G, D_E), jnp.float32),
        ],
    )
    def k(nf_hbm, nid_hbm, ef_hbm, eid_hbm, pn_hbm, pe_hbm,
          nbuf, nidv, ntbuf, ntidv, ebuf, eidv, acc_n, acc_e):
        wid = lax.axis_index("c") * NS + lax.axis_index("s")
        zero = jnp.zeros((LANES,), jnp.float32)

        def zbody(g, carry):
            acc_e[g, :] = zero
            for j in range(D_N // LANES):
                acc_n[g, pl.ds(j * LANES, LANES)] = zero
            return carry
        lax.fori_loop(0, G, zbody, 0)

        nbase = wid * NODE_CHUNK
        pltpu.sync_copy(nid_hbm.at[pl.ds(nbase, NODE_CHUNK)], nidv)
        pltpu.sync_copy(nf_hbm.at[pl.ds(nbase, NODE_CHUNK)], nbuf)

        def nbody(grp, carry):
            i0 = grp * LANES
            gids = nidv[pl.ds(i0, LANES)]
            for l in range(LANES):
                g = gids[l]
                for j in range(D_N // LANES):
                    plsc.addupdate(acc_n.at[g, pl.ds(j * LANES, LANES)],
                                   nbuf[i0 + l, pl.ds(j * LANES, LANES)])
            return carry
        lax.fori_loop(0, NODE_CHUNK // LANES, nbody, 0)

        @pl.when(wid < NODE_TAIL_GROUPS)
        def _tail():
            tb = NW * NODE_CHUNK + wid * LANES
            pltpu.sync_copy(nid_hbm.at[pl.ds(tb, LANES)], ntidv)
            pltpu.sync_copy(nf_hbm.at[pl.ds(tb, LANES)], ntbuf)
            gids = ntidv[...]
            for l in range(LANES):
                g = gids[l]
                for j in range(D_N // LANES):
                    plsc.addupdate(acc_n.at[g, pl.ds(j * LANES, LANES)],
                                   ntbuf[l, pl.ds(j * LANES, LANES)])

        ebase = wid * EDGE_PER_W
        pltpu.sync_copy(eid_hbm.at[pl.ds(ebase, EDGE_PER_W)], eidv)
        for c in range(N_ECHUNKS):
            pltpu.sync_copy(ef_hbm.at[pl.ds(ebase + c * E_CHUNK, E_CHUNK)], ebuf)

            def ebody(grp, carry, c=c):
                i0 = grp * LANES
                gids = eidv[pl.ds(c * E_CHUNK + i0, LANES)]
                for l in range(LANES):
                    g = gids[l]
                    plsc.addupdate(acc_e.at[g], ebuf[i0 + l, :])
                return carry
            lax.fori_loop(0, E_CHUNK // LANES, ebody, 0)

        pltpu.sync_copy(acc_n, pn_hbm.at[wid])
        pltpu.sync_copy(acc_e, pe_hbm.at[wid])

    return k(node_feat, node_ids, edge_feat, edge_ids)


def _combine_body(pn_ref, pe_ref, out_ref):
    out_ref[:, :D_N] = jnp.sum(pn_ref[...], axis=0)
    out_ref[:, D_N:] = jnp.sum(pe_ref[...], axis=0)


def kernel(node_feat, node_graph_ids, edge_feat, edge_graph_ids, num_graphs):
    del num_graphs
    pn, pe = _sc_partials(node_feat, node_graph_ids.astype(jnp.int32),
                          edge_feat, edge_graph_ids.astype(jnp.int32))
    return pl.pallas_call(
        _combine_body,
        out_shape=jax.ShapeDtypeStruct((G, D_N + D_E), jnp.float32),
    )(pn, pe)
